# baseline (speedup 1.0000x reference)
import functools

import jax
import jax.numpy as jnp
from jax import lax
from jax.experimental import pallas as pl
from jax.experimental.pallas import tpu as pltpu
from jax.experimental.pallas import tpu_sc as plsc


@functools.lru_cache(maxsize=None)
def _build_gather(b_total: int, embed: int):
    info = plsc.get_sparse_core_info()
    nc, ns = info.num_cores, info.num_subcores
    nw = nc * ns
    b_per_w = b_total // nw
    assert b_per_w * nw == b_total
    nbuf = 2
    chunk = 800
    while b_per_w % (chunk * nbuf) != 0:
        chunk //= 2
    n_groups = b_per_w // (chunk * nbuf)

    mesh = plsc.VectorSubcoreMesh(core_axis_name="c", subcore_axis_name="s")

    @functools.partial(
        pl.kernel,
        mesh=mesh,
        out_type=jax.ShapeDtypeStruct((b_total, 2 * embed), jnp.float32),
        scratch_types=[
            [pltpu.VMEM((chunk,), jnp.int32) for _ in range(nbuf)],
            [pltpu.VMEM((chunk, embed), jnp.float32) for _ in range(nbuf)],
            [pltpu.SemaphoreType.DMA for _ in range(nbuf)],
            [pltpu.SemaphoreType.DMA for _ in range(nbuf)],
        ],
        compiler_params=pltpu.CompilerParams(use_tc_tiling_on_sc=False,
                                             needs_layout_passes=False),
    )
    def gather_kernel(idx_hbm, table_hbm, out_hbm, idx_v, rows_v, sem_g, sem_o):
        wid = lax.axis_index("s") * nc + lax.axis_index("c")
        base0 = wid * b_per_w

        def out_slice(base):
            return out_hbm.at[pl.ds(base, chunk), pl.ds(0, embed)]

        for b in range(nbuf):
            base = base0 + b * chunk
            pltpu.sync_copy(idx_hbm.at[pl.ds(base, chunk)], idx_v[b])
            pltpu.async_copy(table_hbm.at[idx_v[b]], rows_v[b], sem_g[b])

        def body(g, carry):
            for b in range(nbuf):
                base = base0 + (g * nbuf + b) * chunk
                pltpu.make_async_copy(
                    table_hbm.at[idx_v[b]], rows_v[b], sem_g[b]).wait()
                pltpu.async_copy(rows_v[b], out_slice(base), sem_o[b])

                @pl.when(g < n_groups - 1)
                def _():
                    nbase = base + nbuf * chunk
                    pltpu.sync_copy(idx_hbm.at[pl.ds(nbase, chunk)], idx_v[b])
                    pltpu.make_async_copy(
                        rows_v[b], out_slice(base), sem_o[b]).wait()
                    pltpu.async_copy(table_hbm.at[idx_v[b]], rows_v[b],
                                     sem_g[b])

            return carry

        lax.fori_loop(0, n_groups, body, 0)

        for b in range(nbuf):
            base = base0 + b * chunk
            pltpu.make_async_copy(rows_v[b], out_slice(base), sem_o[b]).wait()

    return gather_kernel


def kernel(input_tensor, embedding_table):
    bsz, hist = input_tensor.shape
    vocab, embed = embedding_table.shape
    idx = input_tensor.reshape(-1).astype(jnp.int32)
    out128 = _build_gather(bsz * hist, embed)(idx, embedding_table)
    return out128[:, :embed].reshape(bsz, hist, embed)
